# baseline (speedup 1.0000x reference)
import functools

import jax
import jax.numpy as jnp
from jax import lax
from jax.experimental import pallas as pl
from jax.experimental.pallas import tpu as pltpu
from jax.experimental.pallas import tpu_sc as plsc

EMBED_DIM = 32
NUM_CORES = 2
NUM_SUBCORES = 16
NUM_WORKERS = NUM_CORES * NUM_SUBCORES

CHUNK = 800
NBUF = 4
LA = NBUF - 1


@functools.lru_cache(maxsize=None)
def _make_gather(n_idx):
    b_per_w = n_idx // NUM_WORKERS
    n_chunks = b_per_w // CHUNK
    n_grp = n_chunks // NBUF
    assert b_per_w % CHUNK == 0 and n_chunks % NBUF == 0
    mesh = plsc.VectorSubcoreMesh(core_axis_name="c", subcore_axis_name="s")

    scratch = (
        [pltpu.VMEM((CHUNK,), jnp.int32) for _ in range(NBUF)]
        + [pltpu.VMEM((CHUNK, EMBED_DIM), jnp.float32) for _ in range(NBUF)]
        + [pltpu.SemaphoreType.DMA for _ in range(2 * NBUF)]
    )

    @functools.partial(
        pl.kernel,
        mesh=mesh,
        compiler_params=pltpu.CompilerParams(use_tc_tiling_on_sc=False),
        out_type=jax.ShapeDtypeStruct((n_idx, EMBED_DIM), jnp.float32),
        scratch_types=scratch,
    )
    def gather_kernel(idx_hbm, table_hbm, out_hbm, *sc):
        idx_bufs = sc[0:NBUF]
        row_bufs = sc[NBUF:2 * NBUF]
        gsem = sc[2 * NBUF:3 * NBUF]
        wsem = sc[3 * NBUF:4 * NBUF]
        wid = lax.axis_index("s") * NUM_CORES + lax.axis_index("c")
        base = wid * b_per_w

        def launch(chunk, b):
            off = base + chunk * CHUNK
            pltpu.sync_copy(idx_hbm.at[pl.ds(off, CHUNK)], idx_bufs[b])
            pltpu.async_copy(table_hbm.at[idx_bufs[b]], row_bufs[b], gsem[b])

        def gather_wait(b):
            pltpu.make_async_copy(table_hbm.at[idx_bufs[b]], row_bufs[b],
                                  gsem[b]).wait()

        def wb_start(chunk, b):
            off = base + chunk * CHUNK
            pltpu.async_copy(row_bufs[b], out_hbm.at[pl.ds(off, CHUNK)],
                             wsem[b])

        def wb_wait(b):
            pltpu.make_async_copy(row_bufs[b], out_hbm.at[pl.ds(base, CHUNK)],
                                  wsem[b]).wait()

        for t in range(LA):
            launch(t, t % NBUF)

        @pl.loop(0, n_grp)
        def _(grp):
            for b in range(NBUF):
                c = grp * NBUF + b
                bg = (b + LA) % NBUF

                if b == 0:
                    @pl.when(grp > 0)
                    def _():
                        wb_wait(bg)
                    launch(c + LA, bg)
                else:
                    @pl.when(grp < n_grp - 1)
                    def _():
                        wb_wait(bg)
                        launch(c + LA, bg)

                gather_wait(b)
                wb_start(c, b)

        for b in range(NBUF):
            wb_wait(b)

    return gather_kernel


def kernel(x, tool_embeddings):
    idx = x.reshape(-1)
    out = _make_gather(idx.shape[0])(idx, tool_embeddings)
    return out.reshape(x.shape + (EMBED_DIM,))
